# baseline (speedup 1.0000x reference)
import functools

import jax
import jax.numpy as jnp
from jax import lax
from jax.experimental import pallas as pl
from jax.experimental.pallas import tpu as pltpu
from jax.experimental.pallas import tpu_sc as plsc

K = 1024
N = 32768
R = 1024
NV = N // 16
NC, NS = 2, 16
NW = NC * NS
RPW = R // NW
NSAMP = 2048
RSAMP = 125

_mesh = plsc.VectorSubcoreMesh(
    core_axis_name="c", subcore_axis_name="s", num_cores=NC, num_subcores=NS
)

_SCRATCH = [
    pltpu.VMEM((N + 16,), jnp.float32),
    pltpu.VMEM((N + 16,), jnp.int32),
    pltpu.VMEM((4096,), jnp.int32),
    pltpu.VMEM((K + 16,), jnp.int32),
    pltpu.VMEM((K,), jnp.float32),
    pltpu.VMEM((NSAMP + 16,), jnp.int32),
    pltpu.VMEM((NSAMP + 16,), jnp.int32),
]

IMIN = -2147483648


def _kmax_body(x_hbm, out_hbm, row_v, tie_v, hist_v, cand_v, orow_v, sa_v, sb_v):
    wid = lax.axis_index("s") * NC + lax.axis_index("c")
    lane = lax.iota(jnp.int32, 16)
    lane16 = lane * 16
    ones = jnp.ones((16,), jnp.int32)
    zeros16 = jnp.zeros((16,), jnp.int32)

    def mono16(vf):
        b = plsc.bitcast(vf, jnp.int32)
        return b ^ (lax.shift_right_arithmetic(b, 31) | IMIN)

    def clear_hist():
        @plsc.parallel_loop(0, 256, unroll=8)
        def _clr(i):
            hist_v[pl.ds(i * 16, 16)] = zeros16

    def find_bstar(need):
        def cond(c):
            _, acc, _ = c
            return acc < need
        def body(c):
            b, acc, _ = c
            b2 = b - 1
            cnt = jnp.sum(hist_v[pl.ds(b2 * 16, 16)])
            return (b2, acc + cnt, cnt)
        b, acc, last = lax.while_loop(
            cond, body, (jnp.int32(256), jnp.int32(0), jnp.int32(0))
        )
        return b, acc - last, last

    def level(load_fn, store_fn, shift, n, need, cand_off):
        clear_hist()
        nv = lax.shift_right_logical(n + 15, 4)

        @plsc.parallel_loop(0, nv, unroll=2)
        def _h(i):
            k = load_fn(i)
            valid = (i * 16 + lane) < n
            b = jnp.bitwise_and(lax.shift_right_logical(k, shift), 0xFF)
            plsc.addupdate_scatter(hist_v, [b * 16 + lane], ones, mask=valid)

        bs, g, n_next = find_bstar(need)
        cov0 = jnp.broadcast_to(cand_off, (16,)).astype(jnp.int32)

        @plsc.parallel_loop(0, nv, unroll=2, carry=(cov0, zeros16))
        def _c(i, carry):
            cov, tov = carry
            k = load_fn(i)
            valid = (i * 16 + lane) < n
            b = jnp.bitwise_and(lax.shift_right_logical(k, shift), 0xFF)
            mgt = (b > bs) & valid
            gi = mgt.astype(jnp.int32)
            plsc.store_scatter(cand_v, [cov + plsc.cumsum(gi) - gi], k, mask=mgt)
            cov = cov + plsc.all_reduce_population_count(mgt)
            if store_fn is not None:
                meq = (b == bs) & valid
                ei = meq.astype(jnp.int32)
                store_fn(tov + plsc.cumsum(ei) - ei, k, meq)
                tov = tov + plsc.all_reduce_population_count(meq)
            return (cov, tov)

        return bs, cand_off + g, n_next, need - g

    def mk_kt(b1, b2, b3, b4):
        return (lax.shift_left(b1, 24) | lax.shift_left(b2, 16)
                | lax.shift_left(b3, 8) | b4)

    def ld_tie(i):
        return tie_v[pl.ds(i * 16, 16)]

    def ld_row_bits(i):
        return plsc.bitcast(row_v[pl.ds(i * 16, 16)], jnp.int32)

    def ld_row_mono(i):
        return mono16(row_v[pl.ds(i * 16, 16)])

    def st_tie(idx, k, m):
        plsc.store_scatter(tie_v, [idx], k, mask=m)

    def st_row(idx, k, m):
        plsc.store_scatter(row_v, [idx], plsc.bitcast(k, jnp.float32), mask=m)

    def ld_sa(i):
        return sa_v[pl.ds(i * 16, 16)]

    def ld_sb(i):
        return sb_v[pl.ds(i * 16, 16)]

    def st_sa(idx, k, m):
        plsc.store_scatter(sa_v, [idx], k, mask=m)

    def st_sb(idx, k, m):
        plsc.store_scatter(sb_v, [idx], k, mask=m)

    def do_row(j, carry):
        r = wid * RPW + j
        pltpu.sync_copy(x_hbm.at[r], row_v.at[pl.ds(0, N)])

        @plsc.parallel_loop(0, NSAMP // 16, unroll=4)
        def _smp(i):
            v = plsc.load_gather(row_v, [i * 256 + lane16])
            sa_v[pl.ds(i * 16, 16)] = mono16(v)

        e1, co, n2, nd = level(ld_sa, st_sb, 24, jnp.int32(NSAMP),
                               jnp.int32(RSAMP), jnp.int32(0))
        e2, co, n3, nd = level(ld_sb, st_sa, 16, n2, nd, co)
        e3, co, n4, nd = level(ld_sa, st_sb, 8, n3, nd, co)
        e4, _, _, _ = level(ld_sb, None, 0, n4, nd, co)
        kt_est = mk_kt(e1, e2, e3, e4)
        kt_u = plsc.bitcast(jnp.broadcast_to(kt_est, (16,)).astype(jnp.int32),
                            jnp.uint32)

        @plsc.parallel_loop(0, NV, unroll=4, carry=zeros16)
        def _pa(i, cov):
            k = mono16(row_v[pl.ds(i * 16, 16)])
            m = plsc.bitcast(k, jnp.uint32) > kt_u
            gi = m.astype(jnp.int32)
            plsc.store_scatter(tie_v, [cov + plsc.cumsum(gi) - gi], k, mask=m)
            return cov + plsc.all_reduce_population_count(m)

        hist_v[pl.ds(0, 16)] = _pa
        ga = hist_v[pl.ds(0, 16)][0]

        def common():
            b1, co, n2, nd = level(ld_tie, st_row, 24, ga,
                                   jnp.int32(K), jnp.int32(0))
            b2, co, n3, nd = level(ld_row_bits, st_tie, 16, n2, nd, co)
            b3, co, n4, nd = level(ld_tie, st_row, 8, n3, nd, co)
            b4, co, _, _ = level(ld_row_bits, None, 0, n4, nd, co)
            return mk_kt(b1, b2, b3, b4), co

        def fallback():
            b1, co, n2, nd = level(ld_row_mono, st_tie, 24, jnp.int32(N),
                                   jnp.int32(K), jnp.int32(0))
            b2, co, n3, nd = level(ld_tie, st_row, 16, n2, nd, co)
            b3, co, n4, nd = level(ld_row_bits, st_tie, 8, n3, nd, co)
            b4, co, _, _ = level(ld_tie, None, 0, n4, nd, co)
            return mk_kt(b1, b2, b3, b4), co

        kt, cand_off = lax.cond(ga >= K, common, fallback)

        @plsc.parallel_loop(0, K // 16, unroll=4)
        def _pad(i):
            idxv = i * 16 + lane
            cur = cand_v[pl.ds(i * 16, 16)]
            cand_v[pl.ds(i * 16, 16)] = jnp.where(idxv < cand_off, cur, kt)

        def vsort_all():
            @plsc.parallel_loop(0, K // 16, unroll=4)
            def _vs(i):
                v = plsc.bitcast(cand_v[pl.ds(i * 16, 16)], jnp.uint32)
                sk, _ = plsc.sort_key_val(v, v, descending=True)
                cand_v[pl.ds(i * 16, 16)] = plsc.bitcast(sk, jnp.int32)

        vsort_all()
        nvk = K // 16
        for t in range(6):
            nb = 1 << t
            if nb == 1:
                @plsc.parallel_loop(0, nvk // 2, unroll=4)
                def _rev1(g):
                    a = 2 * g + 1
                    cand_v[pl.ds(a * 16, 16)] = lax.rev(
                        cand_v[pl.ds(a * 16, 16)], (0,))
            else:
                half = nb // 2
                @plsc.parallel_loop(0, (nvk // (2 * nb)) * half, unroll=4)
                def _revp(p, t=t, nb=nb, half=half):
                    g = lax.shift_right_logical(p, t - 1)
                    i = p & (half - 1)
                    base = g * 2 * nb + nb
                    a = base + i
                    b = base + nb - 1 - i
                    va = lax.rev(cand_v[pl.ds(a * 16, 16)], (0,))
                    vb = lax.rev(cand_v[pl.ds(b * 16, 16)], (0,))
                    cand_v[pl.ds(a * 16, 16)] = vb
                    cand_v[pl.ds(b * 16, 16)] = va
            for sub in range(t, -1, -1):
                s = 1 << sub
                @plsc.parallel_loop(0, nvk // 2, unroll=4)
                def _ce(m, sub=sub, s=s):
                    ia = lax.shift_left(lax.shift_right_logical(m, sub),
                                       sub + 1) | (m & (s - 1))
                    ib = ia + s
                    va = plsc.bitcast(cand_v[pl.ds(ia * 16, 16)], jnp.uint32)
                    vb = plsc.bitcast(cand_v[pl.ds(ib * 16, 16)], jnp.uint32)
                    cand_v[pl.ds(ia * 16, 16)] = plsc.bitcast(
                        jnp.maximum(va, vb), jnp.int32)
                    cand_v[pl.ds(ib * 16, 16)] = plsc.bitcast(
                        jnp.minimum(va, vb), jnp.int32)
            vsort_all()

        @plsc.parallel_loop(0, K // 16, unroll=4)
        def _inv(i):
            k = cand_v[pl.ds(i * 16, 16)]
            b = k ^ (jnp.bitwise_not(lax.shift_right_arithmetic(k, 31)) | IMIN)
            orow_v[pl.ds(i * 16, 16)] = plsc.bitcast(b, jnp.float32)

        pltpu.sync_copy(orow_v, out_hbm.at[r])
        return carry

    lax.fori_loop(0, RPW, do_row, 0)


_kmax_sc = pl.kernel(
    _kmax_body,
    out_type=jax.ShapeDtypeStruct((R, K), jnp.float32),
    mesh=_mesh,
    scratch_types=_SCRATCH,
    compiler_params=pltpu.CompilerParams(needs_layout_passes=False),
)


@jax.jit
def kernel(x):
    x2 = x.reshape(R, N)
    out = _kmax_sc(x2)
    return out.reshape(x.shape[0], x.shape[1], K)


# speedup vs baseline: 1.0947x; 1.0947x over previous
import functools

import jax
import jax.numpy as jnp
from jax import lax
from jax.experimental import pallas as pl
from jax.experimental.pallas import tpu as pltpu
from jax.experimental.pallas import tpu_sc as plsc

K = 1024
N = 32768
R = 1024
NV = N // 16
NC, NS = 2, 16
NW = NC * NS
RPW = R // NW
_mesh = plsc.VectorSubcoreMesh(
    core_axis_name="c", subcore_axis_name="s", num_cores=NC, num_subcores=NS
)

_SCRATCH = [
    pltpu.VMEM((N + 16,), jnp.float32),
    pltpu.VMEM((N + 16,), jnp.float32),
    pltpu.VMEM((N + 16,), jnp.int32),
    pltpu.VMEM((4096,), jnp.int32),
    pltpu.VMEM((K + 16,), jnp.int32),
    pltpu.VMEM((K,), jnp.float32),
    pltpu.SemaphoreType.DMA,
    pltpu.SemaphoreType.DMA,
]

IMIN = -2147483648


def _kmax_body(x_hbm, out_hbm, rowa_v, rowb_v, tie_v, hist_v, cand_v,
               orow_v, sema, semb):
    wid = lax.axis_index("s") * NC + lax.axis_index("c")
    lane = lax.iota(jnp.int32, 16)
    ones = jnp.ones((16,), jnp.int32)
    zeros16 = jnp.zeros((16,), jnp.int32)

    def mono16(vf):
        b = plsc.bitcast(vf, jnp.int32)
        return b ^ (lax.shift_right_arithmetic(b, 31) | IMIN)

    def clear_hist():
        @plsc.parallel_loop(0, 256, unroll=8)
        def _clr(i):
            hist_v[pl.ds(i * 16, 16)] = zeros16

    def find_bstar(need):
        def cond(c):
            _, acc, _ = c
            return acc < need
        def body(c):
            b, acc, _ = c
            b2 = b - 1
            cnt = jnp.sum(hist_v[pl.ds(b2 * 16, 16)])
            return (b2, acc + cnt, cnt)
        b, acc, last = lax.while_loop(
            cond, body, (jnp.int32(256), jnp.int32(0), jnp.int32(0))
        )
        return b, acc - last, last

    def level(load_fn, store_fn, shift, n, need, cand_off):
        clear_hist()
        nv = lax.shift_right_logical(n + 15, 4)

        @plsc.parallel_loop(0, nv, unroll=2)
        def _h(i):
            k = load_fn(i)
            valid = (i * 16 + lane) < n
            b = jnp.bitwise_and(lax.shift_right_logical(k, shift), 0xFF)
            plsc.addupdate_scatter(hist_v, [b * 16 + lane], ones, mask=valid)

        bs, g, n_next = find_bstar(need)
        cov0 = jnp.broadcast_to(cand_off, (16,)).astype(jnp.int32)

        @plsc.parallel_loop(0, nv, unroll=2, carry=(cov0, zeros16))
        def _c(i, carry):
            cov, tov = carry
            k = load_fn(i)
            valid = (i * 16 + lane) < n
            b = jnp.bitwise_and(lax.shift_right_logical(k, shift), 0xFF)
            mgt = (b > bs) & valid
            gi = mgt.astype(jnp.int32)
            plsc.store_scatter(cand_v, [cov + plsc.cumsum(gi) - gi], k, mask=mgt)
            cov = cov + plsc.all_reduce_population_count(mgt)
            if store_fn is not None:
                meq = (b == bs) & valid
                ei = meq.astype(jnp.int32)
                store_fn(tov + plsc.cumsum(ei) - ei, k, meq)
                tov = tov + plsc.all_reduce_population_count(meq)
            return (cov, tov)

        return bs, cand_off + g, n_next, need - g

    def mk_kt(b1, b2, b3, b4):
        return (lax.shift_left(b1, 24) | lax.shift_left(b2, 16)
                | lax.shift_left(b3, 8) | b4)

    def ld_tie(i):
        return tie_v[pl.ds(i * 16, 16)]

    def st_tie(idx, k, m):
        plsc.store_scatter(tie_v, [idx], k, mask=m)

    def process(row_v, r):
        def ld_row_bits(i):
            return plsc.bitcast(row_v[pl.ds(i * 16, 16)], jnp.int32)

        def ld_row_mono(i):
            return mono16(row_v[pl.ds(i * 16, 16)])

        def st_row(idx, k, m):
            plsc.store_scatter(row_v, [idx], plsc.bitcast(k, jnp.float32),
                               mask=m)

        b1, co, n2, nd = level(ld_row_mono, st_tie, 24, jnp.int32(N),
                               jnp.int32(K), jnp.int32(0))
        b2, co, n3, nd = level(ld_tie, st_row, 16, n2, nd, co)
        b3, co, n4, nd = level(ld_row_bits, st_tie, 8, n3, nd, co)
        b4, co, _, _ = level(ld_tie, None, 0, n4, nd, co)
        kt = mk_kt(b1, b2, b3, b4)
        cand_off = co

        @plsc.parallel_loop(0, K // 16, unroll=4)
        def _pad(i):
            idxv = i * 16 + lane
            cur = cand_v[pl.ds(i * 16, 16)]
            cand_v[pl.ds(i * 16, 16)] = jnp.where(idxv < cand_off, cur, kt)

        def vsort_all():
            @plsc.parallel_loop(0, K // 16, unroll=4)
            def _vs(i):
                v = plsc.bitcast(cand_v[pl.ds(i * 16, 16)], jnp.uint32)
                sk, _ = plsc.sort_key_val(v, v, descending=True)
                cand_v[pl.ds(i * 16, 16)] = plsc.bitcast(sk, jnp.int32)

        vsort_all()
        nvk = K // 16
        for t in range(6):
            nb = 1 << t
            if nb == 1:
                @plsc.parallel_loop(0, nvk // 2, unroll=4)
                def _rev1(g):
                    a = 2 * g + 1
                    cand_v[pl.ds(a * 16, 16)] = lax.rev(
                        cand_v[pl.ds(a * 16, 16)], (0,))
            else:
                half = nb // 2
                @plsc.parallel_loop(0, (nvk // (2 * nb)) * half, unroll=4)
                def _revp(p, t=t, nb=nb, half=half):
                    g = lax.shift_right_logical(p, t - 1)
                    i = p & (half - 1)
                    base = g * 2 * nb + nb
                    a = base + i
                    b = base + nb - 1 - i
                    va = lax.rev(cand_v[pl.ds(a * 16, 16)], (0,))
                    vb = lax.rev(cand_v[pl.ds(b * 16, 16)], (0,))
                    cand_v[pl.ds(a * 16, 16)] = vb
                    cand_v[pl.ds(b * 16, 16)] = va
            for sub in range(t, -1, -1):
                s = 1 << sub
                @plsc.parallel_loop(0, nvk // 2, unroll=4)
                def _ce(m, sub=sub, s=s):
                    ia = lax.shift_left(lax.shift_right_logical(m, sub),
                                       sub + 1) | (m & (s - 1))
                    ib = ia + s
                    va = plsc.bitcast(cand_v[pl.ds(ia * 16, 16)], jnp.uint32)
                    vb = plsc.bitcast(cand_v[pl.ds(ib * 16, 16)], jnp.uint32)
                    cand_v[pl.ds(ia * 16, 16)] = plsc.bitcast(
                        jnp.maximum(va, vb), jnp.int32)
                    cand_v[pl.ds(ib * 16, 16)] = plsc.bitcast(
                        jnp.minimum(va, vb), jnp.int32)
            vsort_all()

        @plsc.parallel_loop(0, K // 16, unroll=4)
        def _inv(i):
            k = cand_v[pl.ds(i * 16, 16)]
            b = k ^ (jnp.bitwise_not(lax.shift_right_arithmetic(k, 31)) | IMIN)
            orow_v[pl.ds(i * 16, 16)] = plsc.bitcast(b, jnp.float32)

        pltpu.sync_copy(orow_v, out_hbm.at[r])

    def fetch(r, row_ref, sem):
        return pltpu.make_async_copy(x_hbm.at[r], row_ref.at[pl.ds(0, N)], sem)

    base = wid * RPW
    fetch(base, rowa_v, sema).start()

    def pair(j, carry):
        r0 = base + 2 * j
        fetch(r0, rowa_v, sema).wait()
        fetch(r0 + 1, rowb_v, semb).start()
        process(rowa_v, r0)
        fetch(r0 + 1, rowb_v, semb).wait()

        @pl.when(j < RPW // 2 - 1)
        def _():
            fetch(r0 + 2, rowa_v, sema).start()

        process(rowb_v, r0 + 1)
        return carry

    lax.fori_loop(0, RPW // 2, pair, 0)


_kmax_sc = pl.kernel(
    _kmax_body,
    out_type=jax.ShapeDtypeStruct((R, K), jnp.float32),
    mesh=_mesh,
    scratch_types=_SCRATCH,
    compiler_params=pltpu.CompilerParams(needs_layout_passes=False),
)


@jax.jit
def kernel(x):
    x2 = x.reshape(R, N)
    out = _kmax_sc(x2)
    return out.reshape(x.shape[0], x.shape[1], K)


# speedup vs baseline: 1.5650x; 1.4296x over previous
import functools

import jax
import jax.numpy as jnp
from jax import lax
from jax.experimental import pallas as pl
from jax.experimental.pallas import tpu as pltpu
from jax.experimental.pallas import tpu_sc as plsc

K = 1024
N = 32768
R = 1024
NV = N // 16
NC, NS = 2, 16
NW = NC * NS
RPW = R // NW
_mesh = plsc.VectorSubcoreMesh(
    core_axis_name="c", subcore_axis_name="s", num_cores=NC, num_subcores=NS
)

_SCRATCH = [
    pltpu.VMEM((N + 16,), jnp.float32),
    pltpu.VMEM((N + 16,), jnp.float32),
    pltpu.VMEM((N + 16,), jnp.int32),
    pltpu.VMEM((4096,), jnp.int32),
    pltpu.VMEM((256,), jnp.int32),
    pltpu.VMEM((K + 16,), jnp.int32),
    pltpu.VMEM((K,), jnp.float32),
    pltpu.SemaphoreType.DMA,
    pltpu.SemaphoreType.DMA,
]

IMIN = -2147483648


def _kmax_body(x_hbm, out_hbm, rowa_v, rowb_v, tie_v, hist_v, hb_v, cand_v,
               orow_v, sema, semb):
    wid = lax.axis_index("s") * NC + lax.axis_index("c")
    lane = lax.iota(jnp.int32, 16)
    ones = jnp.ones((16,), jnp.int32)
    zeros16 = jnp.zeros((16,), jnp.int32)

    def mono16(vf):
        b = plsc.bitcast(vf, jnp.int32)
        return b ^ (lax.shift_right_arithmetic(b, 31) | IMIN)

    def clear_hist():
        @plsc.parallel_loop(0, 256, unroll=8)
        def _clr(i):
            hist_v[pl.ds(i * 16, 16)] = zeros16

        @plsc.parallel_loop(0, 16, unroll=4)
        def _clrg(i):
            hb_v[pl.ds(i * 16, 16)] = zeros16

    def find_bstar(need):
        def gcond(c):
            _, acc, _ = c
            return acc < need
        def gbody(c):
            gc, acc, _ = c
            g2 = gc - 1
            cnt = jnp.sum(hb_v[pl.ds(g2 * 16, 16)])
            return (g2, acc + cnt, cnt)
        gstar, gacc, glast = lax.while_loop(
            gcond, gbody, (jnp.int32(16), jnp.int32(0), jnp.int32(0))
        )
        def cond(c):
            _, acc, _ = c
            return acc < need
        def body(c):
            b, acc, _ = c
            b2 = b - 1
            cnt = jnp.sum(hist_v[pl.ds(b2 * 16, 16)])
            return (b2, acc + cnt, cnt)
        b, acc, last = lax.while_loop(
            cond, body, ((gstar + 1) * 16, gacc - glast, jnp.int32(0))
        )
        return b, acc - last, last

    def level(load_fn, store_fn, shift, n, need, cand_off):
        clear_hist()
        nv = lax.shift_right_logical(n + 15, 4)

        @plsc.parallel_loop(0, nv, unroll=4)
        def _h(i):
            k = load_fn(i)
            valid = (i * 16 + lane) < n
            b = jnp.bitwise_and(lax.shift_right_logical(k, shift), 0xFF)
            plsc.addupdate_scatter(hist_v, [b * 16 + lane], ones, mask=valid)
            gb = lax.shift_right_logical(b, 4)
            plsc.addupdate_scatter(hb_v, [gb * 16 + lane], ones, mask=valid)

        bs, g, n_next = find_bstar(need)
        cov0 = jnp.broadcast_to(cand_off, (16,)).astype(jnp.int32)

        @plsc.parallel_loop(0, nv, unroll=4, carry=(cov0, zeros16))
        def _c(i, carry):
            cov, tov = carry
            k = load_fn(i)
            valid = (i * 16 + lane) < n
            b = jnp.bitwise_and(lax.shift_right_logical(k, shift), 0xFF)
            mgt = (b > bs) & valid
            gi = mgt.astype(jnp.int32)
            plsc.store_scatter(cand_v, [cov + plsc.cumsum(gi) - gi], k, mask=mgt)
            cov = cov + plsc.all_reduce_population_count(mgt)
            if store_fn is not None:
                meq = (b == bs) & valid
                ei = meq.astype(jnp.int32)
                store_fn(tov + plsc.cumsum(ei) - ei, k, meq)
                tov = tov + plsc.all_reduce_population_count(meq)
            return (cov, tov)

        return bs, cand_off + g, n_next, need - g

    def mk_kt(b1, b2, b3, b4):
        return (lax.shift_left(b1, 24) | lax.shift_left(b2, 16)
                | lax.shift_left(b3, 8) | b4)

    def ld_tie(i):
        return tie_v[pl.ds(i * 16, 16)]

    def st_tie(idx, k, m):
        plsc.store_scatter(tie_v, [idx], k, mask=m)

    def process(row_v, r):
        def ld_row_bits(i):
            return plsc.bitcast(row_v[pl.ds(i * 16, 16)], jnp.int32)

        def ld_row_mono(i):
            return mono16(row_v[pl.ds(i * 16, 16)])

        def st_row(idx, k, m):
            plsc.store_scatter(row_v, [idx], plsc.bitcast(k, jnp.float32),
                               mask=m)

        b1, co, n2, nd = level(ld_row_mono, st_tie, 24, jnp.int32(N),
                               jnp.int32(K), jnp.int32(0))
        b2, co, n3, nd = level(ld_tie, st_row, 16, n2, nd, co)
        b3, co, n4, nd = level(ld_row_bits, st_tie, 8, n3, nd, co)
        b4, co, _, _ = level(ld_tie, None, 0, n4, nd, co)
        kt = mk_kt(b1, b2, b3, b4)
        cand_off = co

        @plsc.parallel_loop(0, K // 16, unroll=4)
        def _pad(i):
            idxv = i * 16 + lane
            cur = cand_v[pl.ds(i * 16, 16)]
            cand_v[pl.ds(i * 16, 16)] = jnp.where(idxv < cand_off, cur, kt)

        def vsort_all():
            @plsc.parallel_loop(0, K // 16, unroll=4)
            def _vs(i):
                v = plsc.bitcast(cand_v[pl.ds(i * 16, 16)], jnp.uint32)
                sk, _ = plsc.sort_key_val(v, v, descending=True)
                cand_v[pl.ds(i * 16, 16)] = plsc.bitcast(sk, jnp.int32)

        vsort_all()
        nvk = K // 16
        for t in range(6):
            nb = 1 << t
            if nb == 1:
                @plsc.parallel_loop(0, nvk // 2, unroll=4)
                def _rev1(g):
                    a = 2 * g + 1
                    cand_v[pl.ds(a * 16, 16)] = lax.rev(
                        cand_v[pl.ds(a * 16, 16)], (0,))
            else:
                half = nb // 2
                @plsc.parallel_loop(0, (nvk // (2 * nb)) * half, unroll=4)
                def _revp(p, t=t, nb=nb, half=half):
                    g = lax.shift_right_logical(p, t - 1)
                    i = p & (half - 1)
                    base = g * 2 * nb + nb
                    a = base + i
                    b = base + nb - 1 - i
                    va = lax.rev(cand_v[pl.ds(a * 16, 16)], (0,))
                    vb = lax.rev(cand_v[pl.ds(b * 16, 16)], (0,))
                    cand_v[pl.ds(a * 16, 16)] = vb
                    cand_v[pl.ds(b * 16, 16)] = va
            for sub in range(t, -1, -1):
                s = 1 << sub
                @plsc.parallel_loop(0, nvk // 2, unroll=4)
                def _ce(m, sub=sub, s=s):
                    ia = lax.shift_left(lax.shift_right_logical(m, sub),
                                       sub + 1) | (m & (s - 1))
                    ib = ia + s
                    va = plsc.bitcast(cand_v[pl.ds(ia * 16, 16)], jnp.uint32)
                    vb = plsc.bitcast(cand_v[pl.ds(ib * 16, 16)], jnp.uint32)
                    cand_v[pl.ds(ia * 16, 16)] = plsc.bitcast(
                        jnp.maximum(va, vb), jnp.int32)
                    cand_v[pl.ds(ib * 16, 16)] = plsc.bitcast(
                        jnp.minimum(va, vb), jnp.int32)
            vsort_all()

        @plsc.parallel_loop(0, K // 16, unroll=4)
        def _inv(i):
            k = cand_v[pl.ds(i * 16, 16)]
            b = k ^ (jnp.bitwise_not(lax.shift_right_arithmetic(k, 31)) | IMIN)
            orow_v[pl.ds(i * 16, 16)] = plsc.bitcast(b, jnp.float32)

        pltpu.sync_copy(orow_v, out_hbm.at[r])

    def fetch(r, row_ref, sem):
        return pltpu.make_async_copy(x_hbm.at[r], row_ref.at[pl.ds(0, N)], sem)

    base = wid * RPW
    fetch(base, rowa_v, sema).start()

    def pair(j, carry):
        r0 = base + 2 * j
        fetch(r0, rowa_v, sema).wait()
        fetch(r0 + 1, rowb_v, semb).start()
        process(rowa_v, r0)
        fetch(r0 + 1, rowb_v, semb).wait()

        @pl.when(j < RPW // 2 - 1)
        def _():
            fetch(r0 + 2, rowa_v, sema).start()

        process(rowb_v, r0 + 1)
        return carry

    lax.fori_loop(0, RPW // 2, pair, 0)


_kmax_sc = pl.kernel(
    _kmax_body,
    out_type=jax.ShapeDtypeStruct((R, K), jnp.float32),
    mesh=_mesh,
    scratch_types=_SCRATCH,
    compiler_params=pltpu.CompilerParams(needs_layout_passes=False),
)


@jax.jit
def kernel(x):
    x2 = x.reshape(R, N)
    out = _kmax_sc(x2)
    return out.reshape(x.shape[0], x.shape[1], K)


# speedup vs baseline: 1.7491x; 1.1177x over previous
import functools

import jax
import jax.numpy as jnp
from jax import lax
from jax.experimental import pallas as pl
from jax.experimental.pallas import tpu as pltpu
from jax.experimental.pallas import tpu_sc as plsc

K = 1024
N = 32768
R = 1024
NV = N // 16
NC, NS = 2, 16
NW = NC * NS
RPW = R // NW
NSAMP = 2048
RSAMP = 125
_mesh = plsc.VectorSubcoreMesh(
    core_axis_name="c", subcore_axis_name="s", num_cores=NC, num_subcores=NS
)

_SCRATCH = [
    pltpu.VMEM((N + 16,), jnp.float32),
    pltpu.VMEM((N + 16,), jnp.int32),
    pltpu.VMEM((4096,), jnp.int32),
    pltpu.VMEM((256,), jnp.int32),
    pltpu.VMEM((K + 16,), jnp.int32),
    pltpu.VMEM((K,), jnp.float32),
    pltpu.VMEM((2064,), jnp.int32),
    pltpu.VMEM((2064,), jnp.int32),
]

IMIN = -2147483648


def _kmax_body(x_hbm, out_hbm, rowa_v, tie_v, hist_v, hb_v, cand_v,
               orow_v, sa_v, sb_v):
    wid = lax.axis_index("s") * NC + lax.axis_index("c")
    lane = lax.iota(jnp.int32, 16)
    lane16 = lane * 16
    ones = jnp.ones((16,), jnp.int32)
    zeros16 = jnp.zeros((16,), jnp.int32)

    def mono16(vf):
        b = plsc.bitcast(vf, jnp.int32)
        return b ^ (lax.shift_right_arithmetic(b, 31) | IMIN)

    def clear_hist():
        @plsc.parallel_loop(0, 256, unroll=8)
        def _clr(i):
            hist_v[pl.ds(i * 16, 16)] = zeros16

        @plsc.parallel_loop(0, 16, unroll=4)
        def _clrg(i):
            hb_v[pl.ds(i * 16, 16)] = zeros16

    def find_bstar(need):
        def gcond(c):
            _, acc, _ = c
            return acc < need
        def gbody(c):
            gc, acc, _ = c
            g2 = gc - 1
            cnt = jnp.sum(hb_v[pl.ds(g2 * 16, 16)])
            return (g2, acc + cnt, cnt)
        gstar, gacc, glast = lax.while_loop(
            gcond, gbody, (jnp.int32(16), jnp.int32(0), jnp.int32(0))
        )
        def cond(c):
            _, acc, _ = c
            return acc < need
        def body(c):
            b, acc, _ = c
            b2 = b - 1
            cnt = jnp.sum(hist_v[pl.ds(b2 * 16, 16)])
            return (b2, acc + cnt, cnt)
        b, acc, last = lax.while_loop(
            cond, body, ((gstar + 1) * 16, gacc - glast, jnp.int32(0))
        )
        return b, acc - last, last

    def level(load_fn, store_fn, shift, n, need, cand_off):
        clear_hist()
        nv = lax.shift_right_logical(n + 15, 4)

        @plsc.parallel_loop(0, nv, unroll=4)
        def _h(i):
            k = load_fn(i)
            valid = (i * 16 + lane) < n
            b = jnp.bitwise_and(lax.shift_right_logical(k, shift), 0xFF)
            plsc.addupdate_scatter(hist_v, [b * 16 + lane], ones, mask=valid)
            gb = lax.shift_right_logical(b, 4)
            plsc.addupdate_scatter(hb_v, [gb * 16 + lane], ones, mask=valid)

        bs, g, n_next = find_bstar(need)
        cov0 = jnp.broadcast_to(cand_off, (16,)).astype(jnp.int32)

        @plsc.parallel_loop(0, nv, unroll=4, carry=(cov0, zeros16))
        def _c(i, carry):
            cov, tov = carry
            k = load_fn(i)
            valid = (i * 16 + lane) < n
            b = jnp.bitwise_and(lax.shift_right_logical(k, shift), 0xFF)
            mgt = (b > bs) & valid
            gi = mgt.astype(jnp.int32)
            plsc.store_scatter(cand_v, [cov + plsc.cumsum(gi) - gi], k, mask=mgt)
            cov = cov + plsc.all_reduce_population_count(mgt)
            if store_fn is not None:
                meq = (b == bs) & valid
                ei = meq.astype(jnp.int32)
                store_fn(tov + plsc.cumsum(ei) - ei, k, meq)
                tov = tov + plsc.all_reduce_population_count(meq)
            return (cov, tov)

        return bs, cand_off + g, n_next, need - g

    def mk_kt(b1, b2, b3, b4):
        return (lax.shift_left(b1, 24) | lax.shift_left(b2, 16)
                | lax.shift_left(b3, 8) | b4)

    def ld_tie(i):
        return tie_v[pl.ds(i * 16, 16)]

    def st_tie(idx, k, m):
        plsc.store_scatter(tie_v, [idx], k, mask=m)

    def ld_sa(i):
        return sa_v[pl.ds(i * 16, 16)]

    def ld_sb(i):
        return sb_v[pl.ds(i * 16, 16)]

    def st_sa(idx, k, m):
        plsc.store_scatter(sa_v, [idx], k, mask=m)

    def st_sb(idx, k, m):
        plsc.store_scatter(sb_v, [idx], k, mask=m)

    def process(row_v, r):
        def ld_row_bits(i):
            return plsc.bitcast(row_v[pl.ds(i * 16, 16)], jnp.int32)

        def ld_row_mono(i):
            return mono16(row_v[pl.ds(i * 16, 16)])

        def st_row(idx, k, m):
            plsc.store_scatter(row_v, [idx], plsc.bitcast(k, jnp.float32),
                               mask=m)

        @plsc.parallel_loop(0, NSAMP // 16, unroll=4)
        def _smp(i):
            v = plsc.load_gather(row_v, [i * 256 + lane16])
            sa_v[pl.ds(i * 16, 16)] = mono16(v)

        e1, co, n2, nd = level(ld_sa, st_sb, 24, jnp.int32(NSAMP),
                               jnp.int32(RSAMP), jnp.int32(0))
        e2, co, n3, nd = level(ld_sb, st_sa, 16, n2, nd, co)
        e3, co, n4, nd = level(ld_sa, st_sb, 8, n3, nd, co)
        e4, _, _, _ = level(ld_sb, None, 0, n4, nd, co)
        kt_est = mk_kt(e1, e2, e3, e4)
        kt_u = plsc.bitcast(jnp.broadcast_to(kt_est, (16,)).astype(jnp.int32),
                            jnp.uint32)

        @plsc.parallel_loop(0, NV, unroll=4, carry=zeros16)
        def _pa(i, cov):
            k = mono16(row_v[pl.ds(i * 16, 16)])
            m = plsc.bitcast(k, jnp.uint32) > kt_u
            gi = m.astype(jnp.int32)
            plsc.store_scatter(tie_v, [cov + plsc.cumsum(gi) - gi], k, mask=m)
            return cov + plsc.all_reduce_population_count(m)

        hist_v[pl.ds(0, 16)] = _pa
        ga = hist_v[pl.ds(0, 16)][0]

        def common():
            b1, co, n2, nd = level(ld_tie, st_row, 24, ga,
                                   jnp.int32(K), jnp.int32(0))
            b2, co, n3, nd = level(ld_row_bits, st_tie, 16, n2, nd, co)
            b3, co, n4, nd = level(ld_tie, st_row, 8, n3, nd, co)
            b4, co, _, _ = level(ld_row_bits, None, 0, n4, nd, co)
            return mk_kt(b1, b2, b3, b4), co

        def fallback():
            b1, co, n2, nd = level(ld_row_mono, st_tie, 24, jnp.int32(N),
                                   jnp.int32(K), jnp.int32(0))
            b2, co, n3, nd = level(ld_tie, st_row, 16, n2, nd, co)
            b3, co, n4, nd = level(ld_row_bits, st_tie, 8, n3, nd, co)
            b4, co, _, _ = level(ld_tie, None, 0, n4, nd, co)
            return mk_kt(b1, b2, b3, b4), co

        kt, cand_off = lax.cond(ga >= K, common, fallback)

        @plsc.parallel_loop(0, K // 16, unroll=4)
        def _pad(i):
            idxv = i * 16 + lane
            cur = cand_v[pl.ds(i * 16, 16)]
            cand_v[pl.ds(i * 16, 16)] = jnp.where(idxv < cand_off, cur, kt)

        def vsort_all():
            @plsc.parallel_loop(0, K // 16, unroll=4)
            def _vs(i):
                v = plsc.bitcast(cand_v[pl.ds(i * 16, 16)], jnp.uint32)
                sk, _ = plsc.sort_key_val(v, v, descending=True)
                cand_v[pl.ds(i * 16, 16)] = plsc.bitcast(sk, jnp.int32)

        vsort_all()
        nvk = K // 16
        for t in range(6):
            nb = 1 << t
            if nb == 1:
                @plsc.parallel_loop(0, nvk // 2, unroll=4)
                def _rev1(g):
                    a = 2 * g + 1
                    cand_v[pl.ds(a * 16, 16)] = lax.rev(
                        cand_v[pl.ds(a * 16, 16)], (0,))
            else:
                half = nb // 2
                @plsc.parallel_loop(0, (nvk // (2 * nb)) * half, unroll=4)
                def _revp(p, t=t, nb=nb, half=half):
                    g = lax.shift_right_logical(p, t - 1)
                    i = p & (half - 1)
                    base = g * 2 * nb + nb
                    a = base + i
                    b = base + nb - 1 - i
                    va = lax.rev(cand_v[pl.ds(a * 16, 16)], (0,))
                    vb = lax.rev(cand_v[pl.ds(b * 16, 16)], (0,))
                    cand_v[pl.ds(a * 16, 16)] = vb
                    cand_v[pl.ds(b * 16, 16)] = va
            for sub in range(t, -1, -1):
                s = 1 << sub
                @plsc.parallel_loop(0, nvk // 2, unroll=4)
                def _ce(m, sub=sub, s=s):
                    ia = lax.shift_left(lax.shift_right_logical(m, sub),
                                       sub + 1) | (m & (s - 1))
                    ib = ia + s
                    va = plsc.bitcast(cand_v[pl.ds(ia * 16, 16)], jnp.uint32)
                    vb = plsc.bitcast(cand_v[pl.ds(ib * 16, 16)], jnp.uint32)
                    cand_v[pl.ds(ia * 16, 16)] = plsc.bitcast(
                        jnp.maximum(va, vb), jnp.int32)
                    cand_v[pl.ds(ib * 16, 16)] = plsc.bitcast(
                        jnp.minimum(va, vb), jnp.int32)
            vsort_all()

        @plsc.parallel_loop(0, K // 16, unroll=4)
        def _inv(i):
            k = cand_v[pl.ds(i * 16, 16)]
            b = k ^ (jnp.bitwise_not(lax.shift_right_arithmetic(k, 31)) | IMIN)
            orow_v[pl.ds(i * 16, 16)] = plsc.bitcast(b, jnp.float32)

        pltpu.sync_copy(orow_v, out_hbm.at[r])

    base = wid * RPW

    def do_row(j, carry):
        r = base + j
        pltpu.sync_copy(x_hbm.at[r], rowa_v.at[pl.ds(0, N)])
        process(rowa_v, r)
        return carry

    lax.fori_loop(0, RPW, do_row, 0)


_kmax_sc = pl.kernel(
    _kmax_body,
    out_type=jax.ShapeDtypeStruct((R, K), jnp.float32),
    mesh=_mesh,
    scratch_types=_SCRATCH,
    compiler_params=pltpu.CompilerParams(needs_layout_passes=False),
)


@jax.jit
def kernel(x):
    x2 = x.reshape(R, N)
    out = _kmax_sc(x2)
    return out.reshape(x.shape[0], x.shape[1], K)


# speedup vs baseline: 2.0361x; 1.1641x over previous
import functools

import jax
import jax.numpy as jnp
from jax import lax
from jax.experimental import pallas as pl
from jax.experimental.pallas import tpu as pltpu
from jax.experimental.pallas import tpu_sc as plsc

K = 1024
N = 32768
R = 1024
NV = N // 16
NC, NS = 2, 16
NW = NC * NS
RPW = R // NW
NSAMP = 2048
RSAMP = 125
_mesh = plsc.VectorSubcoreMesh(
    core_axis_name="c", subcore_axis_name="s", num_cores=NC, num_subcores=NS
)

_SCRATCH = [
    pltpu.VMEM((N + 16,), jnp.float32),
    pltpu.VMEM((N + 16,), jnp.int32),
    pltpu.VMEM((4096,), jnp.int32),
    pltpu.VMEM((256,), jnp.int32),
    pltpu.VMEM((K + 16,), jnp.int32),
    pltpu.VMEM((K,), jnp.float32),
    pltpu.VMEM((2064,), jnp.int32),
    pltpu.VMEM((2064,), jnp.int32),
]

IMIN = -2147483648


def _kmax_body(x_hbm, out_hbm, rowa_v, tie_v, hist_v, hb_v, cand_v,
               orow_v, sa_v, sb_v):
    wid = lax.axis_index("s") * NC + lax.axis_index("c")
    lane = lax.iota(jnp.int32, 16)
    lane16 = lane * 16
    ones = jnp.ones((16,), jnp.int32)
    zeros16 = jnp.zeros((16,), jnp.int32)

    def mono16(vf):
        b = plsc.bitcast(vf, jnp.int32)
        return b ^ (lax.shift_right_arithmetic(b, 31) | IMIN)

    def clear_hist():
        @plsc.parallel_loop(0, 256, unroll=8)
        def _clr(i):
            hist_v[pl.ds(i * 16, 16)] = zeros16

        @plsc.parallel_loop(0, 16, unroll=4)
        def _clrg(i):
            hb_v[pl.ds(i * 16, 16)] = zeros16

    def find_bstar(need):
        def gcond(c):
            _, acc, _ = c
            return acc < need
        def gbody(c):
            gc, acc, _ = c
            g2 = gc - 1
            cnt = jnp.sum(hb_v[pl.ds(g2 * 16, 16)])
            return (g2, acc + cnt, cnt)
        gstar, gacc, glast = lax.while_loop(
            gcond, gbody, (jnp.int32(16), jnp.int32(0), jnp.int32(0))
        )
        def cond(c):
            _, acc, _ = c
            return acc < need
        def body(c):
            b, acc, _ = c
            b2 = b - 1
            cnt = jnp.sum(hist_v[pl.ds(b2 * 16, 16)])
            return (b2, acc + cnt, cnt)
        b, acc, last = lax.while_loop(
            cond, body, ((gstar + 1) * 16, gacc - glast, jnp.int32(0))
        )
        return b, acc - last, last

    def level(load_fn, store_fn, shift, n, need, cand_off):
        clear_hist()
        nv = lax.shift_right_logical(n + 15, 4)

        @plsc.parallel_loop(0, nv, unroll=4)
        def _h(i):
            k = load_fn(i)
            valid = (i * 16 + lane) < n
            b = jnp.bitwise_and(lax.shift_right_logical(k, shift), 0xFF)
            plsc.addupdate_scatter(hist_v, [b * 16 + lane], ones, mask=valid)
            gb = lax.shift_right_logical(b, 4)
            plsc.addupdate_scatter(hb_v, [gb * 16 + lane], ones, mask=valid)

        bs, g, n_next = find_bstar(need)
        cov0 = jnp.broadcast_to(cand_off, (16,)).astype(jnp.int32)

        @plsc.parallel_loop(0, nv, unroll=4, carry=(cov0, zeros16))
        def _c(i, carry):
            cov, tov = carry
            k = load_fn(i)
            valid = (i * 16 + lane) < n
            b = jnp.bitwise_and(lax.shift_right_logical(k, shift), 0xFF)
            mgt = (b > bs) & valid
            gi = mgt.astype(jnp.int32)
            plsc.store_scatter(cand_v, [cov + plsc.cumsum(gi) - gi], k, mask=mgt)
            cov = cov + plsc.all_reduce_population_count(mgt)
            if store_fn is not None:
                meq = (b == bs) & valid
                ei = meq.astype(jnp.int32)
                store_fn(tov + plsc.cumsum(ei) - ei, k, meq)
                tov = tov + plsc.all_reduce_population_count(meq)
            return (cov, tov)

        return bs, cand_off + g, n_next, need - g

    def mk_kt(b1, b2, b3, b4):
        return (lax.shift_left(b1, 24) | lax.shift_left(b2, 16)
                | lax.shift_left(b3, 8) | b4)

    def ld_tie(i):
        return tie_v[pl.ds(i * 16, 16)]

    def st_tie(idx, k, m):
        plsc.store_scatter(tie_v, [idx], k, mask=m)

    def ld_sa(i):
        return sa_v[pl.ds(i * 16, 16)]

    def ld_sb(i):
        return sb_v[pl.ds(i * 16, 16)]

    def st_sa(idx, k, m):
        plsc.store_scatter(sa_v, [idx], k, mask=m)

    def st_sb(idx, k, m):
        plsc.store_scatter(sb_v, [idx], k, mask=m)

    def process(row_v, r):
        def ld_row_bits(i):
            return plsc.bitcast(row_v[pl.ds(i * 16, 16)], jnp.int32)

        def ld_row_mono(i):
            return mono16(row_v[pl.ds(i * 16, 16)])

        def st_row(idx, k, m):
            plsc.store_scatter(row_v, [idx], plsc.bitcast(k, jnp.float32),
                               mask=m)

        @plsc.parallel_loop(0, NSAMP // 16, unroll=4)
        def _smp(i):
            v = plsc.load_gather(row_v, [i * 256 + lane16])
            sa_v[pl.ds(i * 16, 16)] = mono16(v)

        e1, co, n2, nd = level(ld_sa, st_sb, 24, jnp.int32(NSAMP),
                               jnp.int32(RSAMP), jnp.int32(0))
        e2, _, _, _ = level(ld_sb, None, 16, n2, nd, co)
        kt_est = mk_kt(e1, e2, jnp.int32(0), jnp.int32(0))
        kt_u = plsc.bitcast(jnp.broadcast_to(kt_est, (16,)).astype(jnp.int32),
                            jnp.uint32)

        @plsc.parallel_loop(0, NV, unroll=8, carry=zeros16)
        def _pa(i, cov):
            k = mono16(row_v[pl.ds(i * 16, 16)])
            m = plsc.bitcast(k, jnp.uint32) > kt_u
            gi = m.astype(jnp.int32)
            plsc.store_scatter(tie_v, [cov + plsc.cumsum(gi) - gi], k, mask=m)
            return cov + plsc.all_reduce_population_count(m)

        hist_v[pl.ds(0, 16)] = _pa
        ga = hist_v[pl.ds(0, 16)][0]

        def common():
            b1, co, n2, nd = level(ld_tie, st_row, 24, ga,
                                   jnp.int32(K), jnp.int32(0))
            b2, co, n3, nd = level(ld_row_bits, st_tie, 16, n2, nd, co)
            b3, co, n4, nd = level(ld_tie, st_row, 8, n3, nd, co)
            b4, co, _, _ = level(ld_row_bits, None, 0, n4, nd, co)
            return mk_kt(b1, b2, b3, b4), co

        def fallback():
            b1, co, n2, nd = level(ld_row_mono, st_tie, 24, jnp.int32(N),
                                   jnp.int32(K), jnp.int32(0))
            b2, co, n3, nd = level(ld_tie, st_row, 16, n2, nd, co)
            b3, co, n4, nd = level(ld_row_bits, st_tie, 8, n3, nd, co)
            b4, co, _, _ = level(ld_tie, None, 0, n4, nd, co)
            return mk_kt(b1, b2, b3, b4), co

        kt, cand_off = lax.cond(ga >= K, common, fallback)

        @plsc.parallel_loop(0, K // 16, unroll=4)
        def _pad(i):
            idxv = i * 16 + lane
            cur = cand_v[pl.ds(i * 16, 16)]
            cand_v[pl.ds(i * 16, 16)] = jnp.where(idxv < cand_off, cur, kt)

        def vsort_all():
            @plsc.parallel_loop(0, K // 16, unroll=4)
            def _vs(i):
                v = plsc.bitcast(cand_v[pl.ds(i * 16, 16)], jnp.uint32)
                sk, _ = plsc.sort_key_val(v, v, descending=True)
                cand_v[pl.ds(i * 16, 16)] = plsc.bitcast(sk, jnp.int32)

        vsort_all()
        nvk = K // 16
        for t in range(6):
            nb = 1 << t
            if nb == 1:
                @plsc.parallel_loop(0, nvk // 2, unroll=4)
                def _rev1(g):
                    a = 2 * g + 1
                    cand_v[pl.ds(a * 16, 16)] = lax.rev(
                        cand_v[pl.ds(a * 16, 16)], (0,))
            else:
                half = nb // 2
                @plsc.parallel_loop(0, (nvk // (2 * nb)) * half, unroll=4)
                def _revp(p, t=t, nb=nb, half=half):
                    g = lax.shift_right_logical(p, t - 1)
                    i = p & (half - 1)
                    base = g * 2 * nb + nb
                    a = base + i
                    b = base + nb - 1 - i
                    va = lax.rev(cand_v[pl.ds(a * 16, 16)], (0,))
                    vb = lax.rev(cand_v[pl.ds(b * 16, 16)], (0,))
                    cand_v[pl.ds(a * 16, 16)] = vb
                    cand_v[pl.ds(b * 16, 16)] = va
            for sub in range(t, -1, -1):
                s = 1 << sub
                @plsc.parallel_loop(0, nvk // 2, unroll=4)
                def _ce(m, sub=sub, s=s):
                    ia = lax.shift_left(lax.shift_right_logical(m, sub),
                                       sub + 1) | (m & (s - 1))
                    ib = ia + s
                    va = plsc.bitcast(cand_v[pl.ds(ia * 16, 16)], jnp.uint32)
                    vb = plsc.bitcast(cand_v[pl.ds(ib * 16, 16)], jnp.uint32)
                    cand_v[pl.ds(ia * 16, 16)] = plsc.bitcast(
                        jnp.maximum(va, vb), jnp.int32)
                    cand_v[pl.ds(ib * 16, 16)] = plsc.bitcast(
                        jnp.minimum(va, vb), jnp.int32)
            vsort_all()

        @plsc.parallel_loop(0, K // 16, unroll=4)
        def _inv(i):
            k = cand_v[pl.ds(i * 16, 16)]
            b = k ^ (jnp.bitwise_not(lax.shift_right_arithmetic(k, 31)) | IMIN)
            orow_v[pl.ds(i * 16, 16)] = plsc.bitcast(b, jnp.float32)

        pltpu.sync_copy(orow_v, out_hbm.at[r])

    base = wid * RPW

    def do_row(j, carry):
        r = base + j
        pltpu.sync_copy(x_hbm.at[r], rowa_v.at[pl.ds(0, N)])
        process(rowa_v, r)
        return carry

    lax.fori_loop(0, RPW, do_row, 0)


_kmax_sc = pl.kernel(
    _kmax_body,
    out_type=jax.ShapeDtypeStruct((R, K), jnp.float32),
    mesh=_mesh,
    scratch_types=_SCRATCH,
    compiler_params=pltpu.CompilerParams(needs_layout_passes=False),
)


@jax.jit
def kernel(x):
    x2 = x.reshape(R, N)
    out = _kmax_sc(x2)
    return out.reshape(x.shape[0], x.shape[1], K)
